# baseline (speedup 1.0000x reference)
import functools

import jax
import jax.numpy as jnp
from jax import lax
from jax.experimental import pallas as pl
from jax.experimental.pallas import tpu as pltpu
from jax.experimental.pallas import tpu_sc as plsc

_L = 16
_RES = 13
_CAP = 800
_NCHUNK = 25
_LABCHUNK = 1024


def _emb_kernel(tt_hbm, idx_hbm, out_hbm, stage, labv,
                hitl0, hitl1, hitp0, hitp1, posb0, posb1,
                rowb, sem_st, sem_sc0, sem_sc1, *, num_cores, batch, hidden,
                base_cols, extra_cols):
    w = lax.axis_index("s") * num_cores + lax.axis_index("c")
    c0 = base_cols * w + jnp.minimum(w, extra_cols)
    c1 = c0 + base_cols + jnp.where(w < extra_cols, 1, 0)
    iota = lax.iota(jnp.int32, _L)
    trash = jnp.full((_L,), batch, jnp.int32)
    hitl = (hitl0, hitl1)
    hitp = (hitp0, hitp1)
    posb = (posb0, posb1)

    def fire(r):
        for i in range(_RES):
            col = c0 + _RES * r + i

            @pl.when(col < c1)
            def _():
                pltpu.async_copy(
                    tt_hbm.at[:, pl.ds(col * 128, 128)],
                    stage.at[pl.ds(i * 64, 64)], sem_st)

    def drain(r):
        for i in range(_RES):
            col = c0 + _RES * r + i

            @pl.when(col < c1)
            def _():
                pltpu.make_async_copy(
                    tt_hbm.at[:, pl.ds(0, 128)],
                    stage.at[pl.ds(i * 64, 64)], sem_st).wait()

    fire(0)

    pad0 = jnp.broadcast_to((c0 * 128).astype(jnp.int32), (_L,))
    pad1 = jnp.broadcast_to(((c0 + _RES) * 128).astype(jnp.int32), (_L,))
    for g in range(_CAP // _L):
        hitl0[pl.ds(g * _L, _L)] = pad0
        hitl1[pl.ds(g * _L, _L)] = pad1
        hitp0[pl.ds(g * _L, _L)] = trash
        hitp1[pl.ds(g * _L, _L)] = trash

    def scan_chunk(ch, carry):
        pltpu.sync_copy(idx_hbm.at[pl.ds(ch * _LABCHUNK, _LABCHUNK)], labv)

        def scan_vec(v, carry):
            n0, n1 = carry
            lab = labv[pl.ds(v * _L, _L)]
            col = lax.shift_right_logical(lab, 7)
            pos = ch * _LABCHUNK + v * _L + iota
            m = (col >= c0) & (col < c1)
            rb = col >= c0 + _RES
            m0 = m & jnp.logical_not(rb)
            m1 = m & rb
            cnt0 = plsc.all_reduce_population_count(m0)
            cnt1 = plsc.all_reduce_population_count(m1)
            plsc.store_compressed(hitl0.at[pl.ds(n0, _L)], lab, mask=m0)
            plsc.store_compressed(hitp0.at[pl.ds(n0, _L)], pos, mask=m0)
            plsc.store_compressed(hitl1.at[pl.ds(n1, _L)], lab, mask=m1)
            plsc.store_compressed(hitp1.at[pl.ds(n1, _L)], pos, mask=m1)
            return n0 + cnt0[0], n1 + cnt1[0]

        return lax.fori_loop(0, _LABCHUNK // _L, scan_vec, carry)

    n0, n1 = lax.fori_loop(0, batch // _LABCHUNK, scan_chunk,
                           (jnp.int32(0), jnp.int32(0)))

    for r in range(2):
        for k in range(_NCHUNK):
            for j in range(2):
                posb[r].at[k][pl.ds(j * _L, _L)] = (
                    hitp[r][pl.ds(k * 32 + j * _L, _L)])

    def issue_scatter(src, idx_row, parity):
        @pl.when(parity == 0)
        def _():
            pltpu.async_copy(src, out_hbm.at[idx_row], sem_sc0)

        @pl.when(parity == 1)
        def _():
            pltpu.async_copy(src, out_hbm.at[idx_row], sem_sc1)

    def wait_scatter(parity):
        @pl.when(parity == 0)
        def _():
            pltpu.make_async_copy(
                rowb.at[pl.ds(0, 32)], out_hbm.at[posb0.at[0]],
                sem_sc0).wait()

        @pl.when(parity == 1)
        def _():
            pltpu.make_async_copy(
                rowb.at[pl.ds(0, 32)], out_hbm.at[posb0.at[0]],
                sem_sc1).wait()

    def extract_round(r, nh, counts_in):
        drain(r)
        ngroups = lax.div(nh + (_L - 1), jnp.int32(_L))

        def g_body(g, counts):
            niss0, nw0, niss1, nw1 = counts
            parity = (g >> 1) & 1
            pend0 = (parity == 0) & (niss0 > nw0)
            pend1 = (parity == 1) & (niss1 > nw1)

            @pl.when(((g & 1) == 0) & pend0)
            def _():
                wait_scatter(jnp.int32(0))

            @pl.when(((g & 1) == 0) & pend1)
            def _():
                wait_scatter(jnp.int32(1))

            chunk_start = (g & 1) == 0
            nw0 = nw0 + jnp.where(chunk_start & pend0, 1, 0)
            nw1 = nw1 + jnp.where(chunk_start & pend1, 1, 0)

            lvec = hitl[r][pl.ds(g * _L, _L)]
            slotbase = (g & 3) * _L
            for lane in range(_L):
                l = lvec[lane]
                cl = lax.shift_right_logical(l, 7) - (c0 + _RES * r)
                mm = l & 127
                for j in range(4):
                    ridx = cl * 64 + j * _L + iota
                    cidx = jnp.broadcast_to(mm, (_L,))
                    vals = plsc.load_gather(stage, [ridx, cidx])
                    rowb.at[slotbase + lane][pl.ds(j * _L, _L)] = vals

            @pl.when((g & 1) == 1)
            def _():
                issue_scatter(rowb.at[pl.ds(parity * 32, 32)],
                              posb[r].at[g >> 1], parity)

            last = (g & 1) == 1
            niss0 = niss0 + jnp.where(last & (parity == 0), 1, 0)
            niss1 = niss1 + jnp.where(last & (parity == 1), 1, 0)
            return niss0, nw0, niss1, nw1

        counts = lax.fori_loop(0, ngroups, g_body, counts_in)
        niss0, nw0, niss1, nw1 = counts

        tail = (ngroups & 1) != 0
        tparity = (ngroups >> 1) & 1

        @pl.when(tail)
        def _():
            issue_scatter(rowb.at[pl.ds(tparity * 32, 32)],
                          posb[r].at[ngroups >> 1], tparity)

        niss0 = niss0 + jnp.where(tail & (tparity == 0), 1, 0)
        niss1 = niss1 + jnp.where(tail & (tparity == 1), 1, 0)
        return niss0, nw0, niss1, nw1

    counts = extract_round(0, n0, (jnp.int32(0),) * 4)
    fire(1)
    niss0, nw0, niss1, nw1 = extract_round(1, n1, counts)

    def drain0(i, carry):
        wait_scatter(jnp.int32(0))
        return carry

    def drain1(i, carry):
        wait_scatter(jnp.int32(1))
        return carry

    lax.fori_loop(0, niss0 - nw0, drain0, jnp.int32(0))
    lax.fori_loop(0, niss1 - nw1, drain1, jnp.int32(0))


def kernel(labels, embedding_table):
    (batch,) = labels.shape
    rows, hidden = embedding_table.shape
    info = plsc.get_sparse_core_info()
    num_workers = info.num_cores * info.num_subcores
    cols = -(-rows // 128)
    base_cols = cols // num_workers
    extra_cols = cols % num_workers

    tt = embedding_table.T

    mesh = plsc.VectorSubcoreMesh(core_axis_name="c", subcore_axis_name="s")

    emb = pl.kernel(
        functools.partial(
            _emb_kernel,
            num_cores=info.num_cores,
            batch=batch,
            hidden=hidden,
            base_cols=base_cols,
            extra_cols=extra_cols,
        ),
        out_type=jax.ShapeDtypeStruct((batch + 8, 128), jnp.float32),
        mesh=mesh,
        scratch_types=[
            pltpu.VMEM((_RES * 64, 128), jnp.float32),
            pltpu.VMEM((_LABCHUNK,), jnp.int32),
            pltpu.VMEM((_CAP,), jnp.int32),
            pltpu.VMEM((_CAP,), jnp.int32),
            pltpu.VMEM((_CAP,), jnp.int32),
            pltpu.VMEM((_CAP,), jnp.int32),
            pltpu.VMEM((_NCHUNK, 32), jnp.int32),
            pltpu.VMEM((_NCHUNK, 32), jnp.int32),
            pltpu.VMEM((64, 128), jnp.float32),
            pltpu.SemaphoreType.DMA,
            pltpu.SemaphoreType.DMA,
            pltpu.SemaphoreType.DMA,
        ],
        compiler_params=pltpu.CompilerParams(
            use_tc_tiling_on_sc=True, needs_layout_passes=False),
    )
    out = emb(tt, labels.astype(jnp.int32))
    return out[:batch, :hidden]
